# Initial kernel scaffold; baseline (speedup 1.0000x reference)
import jax
import jax.numpy as jnp
from jax.experimental import pallas as pl


def kernel(x, edge_index, W1, b1, W2, b2, W3, b3, Wh, bh):
    raise NotImplementedError("write your pallas kernel here")


# speedup vs baseline: 6.7603x; 6.7603x over previous
import functools

import jax
import jax.numpy as jnp
from jax import lax
from jax.experimental import pallas as pl
from jax.experimental.pallas import tpu as pltpu
from jax.experimental.pallas import tpu_sc as plsc

N = 10000
E = 320000
D = 128
NC = 2
NS = 16
NW = NC * NS
CH = 128
NCHUNK = 80
EPAD = NW * NCHUNK * CH
NPAD = 10240
ZROWS = NPAD // NS
OROWS = N // NS
DEGW = 8

_sc_mesh = plsc.VectorSubcoreMesh(core_axis_name="c", subcore_axis_name="s")


@functools.partial(
    pl.kernel,
    mesh=_sc_mesh,
    out_type=jax.ShapeDtypeStruct((NC, NPAD, DEGW), jnp.float32),
    scratch_types=[
        pltpu.VMEM_SHARED((NPAD, DEGW), jnp.float32),
        pltpu.VMEM((NCHUNK, 2, CH), jnp.int32),
        pltpu.VMEM((CH, DEGW), jnp.float32),
    ],
)
def _deg_sc(edges_t, ones_hbm, zeros_hbm, deg_out, deg_sp, edges_v, ones_v):
    c = lax.axis_index("c")
    s = lax.axis_index("s")
    w = c * NS + s
    pltpu.sync_copy(edges_t.at[w], edges_v)
    pltpu.sync_copy(ones_hbm, ones_v)
    pltpu.sync_copy(zeros_hbm, deg_sp.at[pl.ds(s * ZROWS, ZROWS)])
    plsc.subcore_barrier()

    @pl.loop(0, NCHUNK)
    def _(j):
        pltpu.sync_copy(ones_v, deg_sp.at[edges_v.at[j, 1]], add=True)

    plsc.subcore_barrier()
    pltpu.sync_copy(
        deg_sp.at[pl.ds(s * ZROWS, ZROWS)],
        deg_out.at[c, pl.ds(s * ZROWS, ZROWS)],
    )


@functools.partial(
    pl.kernel,
    mesh=_sc_mesh,
    out_type=jax.ShapeDtypeStruct((NC, NPAD, D), jnp.float32),
    scratch_types=[
        pltpu.VMEM_SHARED((NPAD, D), jnp.float32),
        pltpu.VMEM((2, CH), jnp.int32),
        pltpu.VMEM((2, CH), jnp.int32),
        pltpu.VMEM((CH, D), jnp.float32),
        pltpu.VMEM((CH, D), jnp.float32),
        pltpu.SemaphoreType.DMA,
        pltpu.SemaphoreType.DMA,
        pltpu.SemaphoreType.DMA,
        pltpu.SemaphoreType.DMA,
    ],
)
def _agg_sc(hp, edges_t, zeros_hbm, y_out,
            y_sp, eb_a, eb_b, rows_a, rows_b, sem_a, sem_b, isem_a, isem_b):
    c = lax.axis_index("c")
    s = lax.axis_index("s")
    w = c * NS + s
    pltpu.sync_copy(zeros_hbm, y_sp.at[pl.ds(s * ZROWS, ZROWS)])
    plsc.subcore_barrier()

    pltpu.sync_copy(edges_t.at[w, 0], eb_a)
    pltpu.async_copy(hp.at[eb_a.at[0]], rows_a, sem_a)
    pltpu.async_copy(edges_t.at[w, 1], eb_b, isem_b)

    @pl.loop(0, NCHUNK // 2)
    def _(jj):
        j0 = jj * 2
        more = jj + 1 < NCHUNK // 2
        pltpu.make_async_copy(edges_t.at[w, 0], eb_b, isem_b).wait()
        pltpu.make_async_copy(hp.at[eb_a.at[0]], rows_a, sem_a).wait()
        pltpu.async_copy(hp.at[eb_b.at[0]], rows_b, sem_b)
        pltpu.sync_copy(rows_a, y_sp.at[eb_a.at[1]], add=True)

        @pl.when(more)
        def _():
            pltpu.async_copy(edges_t.at[w, j0 + 2], eb_a, isem_a)

        pltpu.make_async_copy(hp.at[eb_a.at[0]], rows_b, sem_b).wait()

        @pl.when(more)
        def _():
            pltpu.make_async_copy(edges_t.at[w, 0], eb_a, isem_a).wait()
            pltpu.async_copy(hp.at[eb_a.at[0]], rows_a, sem_a)

        pltpu.sync_copy(rows_b, y_sp.at[eb_b.at[1]], add=True)

        @pl.when(more)
        def _():
            pltpu.async_copy(edges_t.at[w, j0 + 3], eb_b, isem_b)

    plsc.subcore_barrier()
    pltpu.sync_copy(
        y_sp.at[pl.ds(s * ZROWS, ZROWS)],
        y_out.at[c, pl.ds(s * ZROWS, ZROWS)],
    )


def _dinv_body(deg_ref, out_ref):
    d = deg_ref[0] + deg_ref[1] + 1.0
    v = lax.rsqrt(d)
    out_ref[...] = v[:N, 0:1]


def _dinv_tc(deg):
    return pl.pallas_call(
        _dinv_body,
        out_shape=jax.ShapeDtypeStruct((N, 1), jnp.float32),
    )(deg)


_RB = 2000
_GRID = N // _RB


def _mm_scale_body(x_ref, w_ref, dinv_ref, out_ref):
    h = jnp.dot(x_ref[...], w_ref[...], preferred_element_type=jnp.float32)
    out_ref[...] = dinv_ref[...] * h


def _mm_scale(x, W, dinv):
    return pl.pallas_call(
        _mm_scale_body,
        grid=(_GRID,),
        in_specs=[
            pl.BlockSpec((_RB, D), lambda i: (i, 0)),
            pl.BlockSpec((D, D), lambda i: (0, 0)),
            pl.BlockSpec((_RB, 1), lambda i: (i, 0)),
        ],
        out_specs=pl.BlockSpec((_RB, D), lambda i: (i, 0)),
        out_shape=jax.ShapeDtypeStruct((N, D), jnp.float32),
    )(x, W, dinv)


def _combine_mm_body(y_ref, hp_ref, dinv_ref, b_ref, w_ref, out_ref):
    t = dinv_ref[...] * (y_ref[0] + y_ref[1] + hp_ref[...]) + b_ref[...]
    t = jnp.maximum(t, 0.0)
    h = jnp.dot(t, w_ref[...], preferred_element_type=jnp.float32)
    out_ref[...] = dinv_ref[...] * h


def _combine_mm(y, hp, dinv, b, W):
    return pl.pallas_call(
        _combine_mm_body,
        grid=(_GRID,),
        in_specs=[
            pl.BlockSpec((NC, _RB, D), lambda i: (0, i, 0)),
            pl.BlockSpec((_RB, D), lambda i: (i, 0)),
            pl.BlockSpec((_RB, 1), lambda i: (i, 0)),
            pl.BlockSpec((1, D), lambda i: (0, 0)),
            pl.BlockSpec((D, D), lambda i: (0, 0)),
        ],
        out_specs=pl.BlockSpec((_RB, D), lambda i: (i, 0)),
        out_shape=jax.ShapeDtypeStruct((N, D), jnp.float32),
    )(y, hp, dinv, b, W)


def _head_body(y_ref, hp_ref, dinv_ref, b_ref, wh_ref, bh_ref, out_ref, acc_ref):
    i = pl.program_id(0)
    ps = jnp.sum(dinv_ref[...] * (y_ref[0] + y_ref[1] + hp_ref[...]),
                 axis=0, keepdims=True)
    acc_ref[...] = jnp.where(i == 0, ps, acc_ref[...] + ps)

    @pl.when(i == _GRID - 1)
    def _():
        g = acc_ref[...] * (1.0 / N) + b_ref[...]
        out_ref[...] = (
            jnp.dot(g, wh_ref[...], preferred_element_type=jnp.float32)
            + bh_ref[...]
        )


def _head(y, hp, dinv, b, Wh, bh):
    return pl.pallas_call(
        _head_body,
        grid=(_GRID,),
        in_specs=[
            pl.BlockSpec((NC, _RB, D), lambda i: (0, i, 0)),
            pl.BlockSpec((_RB, D), lambda i: (i, 0)),
            pl.BlockSpec((_RB, 1), lambda i: (i, 0)),
            pl.BlockSpec((1, D), lambda i: (0, 0)),
            pl.BlockSpec((D, D), lambda i: (0, 0)),
            pl.BlockSpec((1, D), lambda i: (0, 0)),
        ],
        out_specs=pl.BlockSpec((1, D), lambda i: (0, 0)),
        out_shape=jax.ShapeDtypeStruct((1, D), jnp.float32),
        scratch_shapes=[pltpu.VMEM((1, D), jnp.float32)],
    )(y, hp, dinv, b, Wh, bh)


def kernel(x, edge_index, W1, b1, W2, b2, W3, b3, Wh, bh):
    src = edge_index[0].astype(jnp.int32)
    dst = edge_index[1].astype(jnp.int32)
    npad = EPAD - E
    src_t = jnp.concatenate([src, jnp.zeros((npad,), jnp.int32)])
    dst_t = jnp.concatenate([dst, jnp.full((npad,), N, jnp.int32)])
    edges_t = jnp.stack(
        [src_t.reshape(NW, NCHUNK, CH), dst_t.reshape(NW, NCHUNK, CH)], axis=2
    )

    ones = jnp.ones((CH, DEGW), jnp.float32)
    zeros_d = jnp.zeros((ZROWS, DEGW), jnp.float32)
    zeros_y = jnp.zeros((ZROWS, D), jnp.float32)

    deg = _deg_sc(edges_t, ones, zeros_d)
    dinv = _dinv_tc(deg)

    hp1 = _mm_scale(x, W1, dinv)
    y1 = _agg_sc(hp1, edges_t, zeros_y)
    hp2 = _combine_mm(y1, hp1, dinv, b1.reshape(1, D), W2)
    y2 = _agg_sc(hp2, edges_t, zeros_y)
    hp3 = _combine_mm(y2, hp2, dinv, b2.reshape(1, D), W3)
    y3 = _agg_sc(hp3, edges_t, zeros_y)
    return _head(y3, hp3, dinv, b3.reshape(1, D), Wh, bh.reshape(1, D))


# speedup vs baseline: 15.6545x; 2.3156x over previous
import functools

import jax
import jax.numpy as jnp
from jax import lax
from jax.experimental import pallas as pl
from jax.experimental.pallas import tpu as pltpu
from jax.experimental.pallas import tpu_sc as plsc

N = 10000
E = 320000
D = 128
NC = 2
NS = 16
NW = NC * NS
CH = 128
NCHUNK = 84
EPAD = NW * NCHUNK * CH
NPAD = 10016
ZROWS = 640
LROWS = NPAD - 15 * ZROWS
DEGW = 8
NB = 3
NI = 6

_sc_mesh = plsc.VectorSubcoreMesh(core_axis_name="c", subcore_axis_name="s")


NPAD_DEG = 10240


@functools.partial(
    pl.kernel,
    mesh=_sc_mesh,
    out_type=jax.ShapeDtypeStruct((NC, NPAD_DEG, DEGW), jnp.float32),
    scratch_types=[
        pltpu.VMEM_SHARED((NPAD_DEG, DEGW), jnp.float32),
        pltpu.VMEM((NCHUNK, 2, CH), jnp.int32),
        pltpu.VMEM((CH, DEGW), jnp.float32),
    ],
)
def _deg_sc(edges_t, ones_hbm, zeros_hbm, deg_out, deg_sp, edges_v, ones_v):
    c = lax.axis_index("c")
    s = lax.axis_index("s")
    w = c * NS + s
    pltpu.sync_copy(edges_t.at[w], edges_v)
    pltpu.sync_copy(ones_hbm, ones_v)
    pltpu.sync_copy(zeros_hbm, deg_sp.at[pl.ds(s * 640, 640)])
    plsc.subcore_barrier()

    @pl.loop(0, NCHUNK)
    def _(j):
        pltpu.sync_copy(ones_v, deg_sp.at[edges_v.at[j, 1]], add=True)

    plsc.subcore_barrier()
    pltpu.sync_copy(deg_sp.at[pl.ds(s * 640, 640)],
                    deg_out.at[c, pl.ds(s * 640, 640)])


@functools.partial(
    pl.kernel,
    mesh=_sc_mesh,
    out_type=jax.ShapeDtypeStruct((NC, NPAD, D), jnp.float32),
    scratch_types=[pltpu.VMEM_SHARED((NPAD, D), jnp.float32)]
    + [pltpu.VMEM((CH, D), jnp.float32) for _ in range(NB)]
    + [pltpu.VMEM((2, CH), jnp.int32) for _ in range(NI)]
    + [pltpu.SemaphoreType.DMA for _ in range(2 * NB + NI)],
)
def _agg_sc(hp, edges_t, zeros_hbm, y_out, y_sp, *bufs):
    rows = bufs[:NB]
    ebuf = bufs[NB:NB + NI]
    gsem = bufs[NB + NI:NB + NI + NB]
    ssem = bufs[NB + NI + NB:NB + NI + 2 * NB]
    isem = bufs[NB + NI + 2 * NB:]
    c = lax.axis_index("c")
    s = lax.axis_index("s")
    w = c * NS + s

    @pl.when(s < NS - 1)
    def _():
        pltpu.sync_copy(zeros_hbm, y_sp.at[pl.ds(s * ZROWS, ZROWS)])

    @pl.when(s == NS - 1)
    def _():
        pltpu.sync_copy(zeros_hbm.at[pl.ds(0, LROWS)],
                        y_sp.at[pl.ds((NS - 1) * ZROWS, LROWS)])

    plsc.subcore_barrier()

    for k in range(3):
        pltpu.async_copy(edges_t.at[w, k], ebuf[k], isem[k])

    @pl.loop(0, NCHUNK // NI)
    def _(jj):
        j0 = jj * NI
        for k in range(NI):
            j = j0 + k
            p, q = k % NB, k % NI
            pm1, qm1 = (k - 1) % NB, (k - 1) % NI

            @pl.when(j >= NB)
            def _():
                pltpu.make_async_copy(
                    rows[p], y_sp.at[ebuf[(q + 3) % NI].at[1]], ssem[p]
                ).wait()

            pltpu.make_async_copy(edges_t.at[w, j], ebuf[q], isem[q]).wait()
            pltpu.async_copy(hp.at[ebuf[q].at[0]], rows[p], gsem[p])

            @pl.when(j + 3 < NCHUNK)
            def _():
                pltpu.async_copy(edges_t.at[w, j + 3], ebuf[(k + 3) % NI],
                                 isem[(k + 3) % NI])

            @pl.when(j >= 1)
            def _():
                pltpu.make_async_copy(
                    hp.at[ebuf[qm1].at[0]], rows[pm1], gsem[pm1]).wait()
                pltpu.async_copy(rows[pm1], y_sp.at[ebuf[qm1].at[1]],
                                 ssem[pm1], add=True)

    lp, lq = (NCHUNK - 1) % NB, (NCHUNK - 1) % NI
    pltpu.make_async_copy(hp.at[ebuf[lq].at[0]], rows[lp], gsem[lp]).wait()
    pltpu.async_copy(rows[lp], y_sp.at[ebuf[lq].at[1]], ssem[lp], add=True)
    for j in (NCHUNK - 3, NCHUNK - 2, NCHUNK - 1):
        pltpu.make_async_copy(
            rows[j % NB], y_sp.at[ebuf[j % NI].at[1]], ssem[j % NB]).wait()

    plsc.subcore_barrier()

    @pl.when(s < NS - 1)
    def _():
        pltpu.sync_copy(y_sp.at[pl.ds(s * ZROWS, ZROWS)],
                        y_out.at[c, pl.ds(s * ZROWS, ZROWS)])

    @pl.when(s == NS - 1)
    def _():
        pltpu.sync_copy(y_sp.at[pl.ds((NS - 1) * ZROWS, LROWS)],
                        y_out.at[c, pl.ds((NS - 1) * ZROWS, LROWS)])


def _dinv_body(deg_ref, out_ref):
    d = deg_ref[0] + deg_ref[1] + 1.0
    v = lax.rsqrt(d)
    out_ref[...] = v[:N, 0:1]


def _dinv_tc(deg):
    return pl.pallas_call(
        _dinv_body,
        out_shape=jax.ShapeDtypeStruct((N, 1), jnp.float32),
    )(deg)


_RB = 2000
_GRID = N // _RB


def _mm_scale_body(x_ref, w_ref, dinv_ref, out_ref):
    h = jnp.dot(x_ref[...], w_ref[...], preferred_element_type=jnp.float32)
    out_ref[...] = dinv_ref[...] * h


def _mm_scale(x, W, dinv):
    return pl.pallas_call(
        _mm_scale_body,
        grid=(_GRID,),
        in_specs=[
            pl.BlockSpec((_RB, D), lambda i: (i, 0)),
            pl.BlockSpec((D, D), lambda i: (0, 0)),
            pl.BlockSpec((_RB, 1), lambda i: (i, 0)),
        ],
        out_specs=pl.BlockSpec((_RB, D), lambda i: (i, 0)),
        out_shape=jax.ShapeDtypeStruct((N, D), jnp.float32),
    )(x, W, dinv)


def _combine_mm_body(y_ref, hp_ref, dinv_ref, b_ref, w_ref, out_ref):
    t = dinv_ref[...] * (y_ref[0] + y_ref[1] + hp_ref[...]) + b_ref[...]
    t = jnp.maximum(t, 0.0)
    h = jnp.dot(t, w_ref[...], preferred_element_type=jnp.float32)
    out_ref[...] = dinv_ref[...] * h


def _combine_mm(y, hp, dinv, b, W):
    return pl.pallas_call(
        _combine_mm_body,
        grid=(_GRID,),
        in_specs=[
            pl.BlockSpec((NC, _RB, D), lambda i: (0, i, 0)),
            pl.BlockSpec((_RB, D), lambda i: (i, 0)),
            pl.BlockSpec((_RB, 1), lambda i: (i, 0)),
            pl.BlockSpec((1, D), lambda i: (0, 0)),
            pl.BlockSpec((D, D), lambda i: (0, 0)),
        ],
        out_specs=pl.BlockSpec((_RB, D), lambda i: (i, 0)),
        out_shape=jax.ShapeDtypeStruct((N, D), jnp.float32),
    )(y, hp, dinv, b, W)


def _head_body(y_ref, hp_ref, dinv_ref, b_ref, wh_ref, bh_ref, out_ref, acc_ref):
    i = pl.program_id(0)
    ps = jnp.sum(dinv_ref[...] * (y_ref[0] + y_ref[1] + hp_ref[...]),
                 axis=0, keepdims=True)
    acc_ref[...] = jnp.where(i == 0, ps, acc_ref[...] + ps)

    @pl.when(i == _GRID - 1)
    def _():
        g = acc_ref[...] * (1.0 / N) + b_ref[...]
        out_ref[...] = (
            jnp.dot(g, wh_ref[...], preferred_element_type=jnp.float32)
            + bh_ref[...]
        )


def _head(y, hp, dinv, b, Wh, bh):
    return pl.pallas_call(
        _head_body,
        grid=(_GRID,),
        in_specs=[
            pl.BlockSpec((NC, _RB, D), lambda i: (0, i, 0)),
            pl.BlockSpec((_RB, D), lambda i: (i, 0)),
            pl.BlockSpec((_RB, 1), lambda i: (i, 0)),
            pl.BlockSpec((1, D), lambda i: (0, 0)),
            pl.BlockSpec((D, D), lambda i: (0, 0)),
            pl.BlockSpec((1, D), lambda i: (0, 0)),
        ],
        out_specs=pl.BlockSpec((1, D), lambda i: (0, 0)),
        out_shape=jax.ShapeDtypeStruct((1, D), jnp.float32),
        scratch_shapes=[pltpu.VMEM((1, D), jnp.float32)],
    )(y, hp, dinv, b, Wh, bh)


def kernel(x, edge_index, W1, b1, W2, b2, W3, b3, Wh, bh):
    src = edge_index[0].astype(jnp.int32)
    dst = edge_index[1].astype(jnp.int32)
    npad = EPAD - E
    fill = jnp.arange(npad, dtype=jnp.int32) % 16
    src_t = jnp.concatenate([src, fill])
    dst_t = jnp.concatenate([dst, N + fill])
    edges_t = jnp.stack(
        [src_t.reshape(NW, NCHUNK, CH), dst_t.reshape(NW, NCHUNK, CH)], axis=2
    )

    ones = jnp.ones((CH, DEGW), jnp.float32)
    zeros_d = jnp.zeros((640, DEGW), jnp.float32)
    zeros_y = jnp.zeros((ZROWS, D), jnp.float32)

    deg = _deg_sc(edges_t, ones, zeros_d)
    dinv = _dinv_tc(deg)

    hp1 = _mm_scale(x, W1, dinv)
    y1 = _agg_sc(hp1, edges_t, zeros_y)
    hp2 = _combine_mm(y1, hp1, dinv, b1.reshape(1, D), W2)
    y2 = _agg_sc(hp2, edges_t, zeros_y)
    hp3 = _combine_mm(y2, hp2, dinv, b2.reshape(1, D), W3)
    y3 = _agg_sc(hp3, edges_t, zeros_y)
    return _head(y3, hp3, dinv, b3.reshape(1, D), Wh, bh.reshape(1, D))
